# Initial kernel scaffold; baseline (speedup 1.0000x reference)
import jax
import jax.numpy as jnp
from jax.experimental import pallas as pl


def kernel(hidden_states, weight, e_score_correction_bias):
    raise NotImplementedError("write your pallas kernel here")


# speedup vs baseline: 1.3064x; 1.3064x over previous
import functools

import jax
import jax.numpy as jnp
from jax import lax
from jax.experimental import pallas as pl
from jax.experimental.pallas import tpu as pltpu

N_EXPERTS = 64
N_GROUP = 8
GROUP_SIZE = N_EXPERTS // N_GROUP
TOPK_GROUP = 4
TOP_K = 8
SCALE = 2.5
NEG_INF = float("-inf")


def _first_argmax(x, col_iota, width):
    m = jnp.max(x, axis=1, keepdims=True)
    am = jnp.min(jnp.where(x == m, col_iota, width), axis=1, keepdims=True)
    return m, am


def _router_kernel(x_ref, wt_ref, bias_ref, idx_ref, w_ref):
    tb = x_ref.shape[0]
    logits = jax.lax.dot_general(
        x_ref[...], wt_ref[...],
        dimension_numbers=(((1,), (0,)), ((), ())),
        preferred_element_type=jnp.float32,
        precision=jax.lax.Precision.DEFAULT,
    )
    scores = jax.nn.sigmoid(logits)
    sfc = scores + bias_ref[...]

    io8 = lax.broadcasted_iota(jnp.int32, (tb, N_GROUP), 1)
    io_gs = lax.broadcasted_iota(jnp.int32, (tb, GROUP_SIZE), 1)
    io64 = lax.broadcasted_iota(jnp.int32, (tb, N_EXPERTS), 1)
    gid64 = lax.shift_right_logical(io64, 3)

    cols = []
    for g in range(N_GROUP):
        blk = sfc[:, g * GROUP_SIZE:(g + 1) * GROUP_SIZE]
        m1, am = _first_argmax(blk, io_gs, GROUP_SIZE)
        m2 = jnp.max(jnp.where(io_gs == am, NEG_INF, blk), axis=1, keepdims=True)
        cols.append(m1 + m2)
    group_scores = jnp.concatenate(cols, axis=1)

    smask = jnp.zeros((tb, N_EXPERTS), dtype=jnp.bool_)
    gs = group_scores
    for _ in range(TOPK_GROUP):
        _, am = _first_argmax(gs, io8, N_GROUP)
        smask = jnp.logical_or(smask, gid64 == am)
        gs = jnp.where(io8 == am, NEG_INF, gs)

    tmp = jnp.where(smask, sfc, NEG_INF)
    idx_cols = []
    w_cols = []
    for _ in range(TOP_K):
        _, am = _first_argmax(tmp, io64, N_EXPERTS)
        sel = io64 == am
        idx_cols.append(am)
        w_cols.append(jnp.max(jnp.where(sel, scores, NEG_INF), axis=1, keepdims=True))
        tmp = jnp.where(sel, NEG_INF, tmp)
    topk_idx = jnp.concatenate(idx_cols, axis=1)
    topk_w = jnp.concatenate(w_cols, axis=1)

    denom = jnp.sum(topk_w, axis=1, keepdims=True) + 1e-20
    idx_ref[...] = topk_idx
    w_ref[...] = topk_w / denom * SCALE


@jax.jit
def _run(x, weight_t, bias):
    t = x.shape[0]
    tb = 512
    grid = (t // tb,)
    return pl.pallas_call(
        _router_kernel,
        grid=grid,
        in_specs=[
            pl.BlockSpec((tb, x.shape[1]), lambda i: (i, 0)),
            pl.BlockSpec((x.shape[1], N_EXPERTS), lambda i: (0, 0)),
            pl.BlockSpec((1, N_EXPERTS), lambda i: (0, 0)),
        ],
        out_specs=[
            pl.BlockSpec((tb, TOP_K), lambda i: (i, 0)),
            pl.BlockSpec((tb, TOP_K), lambda i: (i, 0)),
        ],
        out_shape=[
            jax.ShapeDtypeStruct((t, TOP_K), jnp.int32),
            jax.ShapeDtypeStruct((t, TOP_K), jnp.float32),
        ],
    )(x, weight_t, bias)


def kernel(hidden_states, weight, e_score_correction_bias):
    bsz, seq_len, h = hidden_states.shape
    x = hidden_states.reshape(-1, h).astype(jnp.float32)
    weight_t = weight.astype(jnp.float32).T
    bias = e_score_correction_bias.reshape(1, N_EXPERTS).astype(jnp.float32)
    topk_idx, topk_weight = _run(x, weight_t, bias)
    return topk_idx, topk_weight


# speedup vs baseline: 5.0809x; 3.8891x over previous
import jax
import jax.numpy as jnp
from jax import lax
from jax.experimental import pallas as pl

N_EXPERTS = 64
N_GROUP = 8
GROUP_SIZE = N_EXPERTS // N_GROUP
TOPK_GROUP = 4
TOP_K = 8
SCALE = 2.5
NEG_INF = float("-inf")


def _first_argmax0(x, row_iota, height):
    m = jnp.max(x, axis=0, keepdims=True)
    am = jnp.min(jnp.where(x == m, row_iota, height), axis=0, keepdims=True)
    return m, am


def _router_kernel(x_ref, w_ref, bias_ref, idx_ref, w_out_ref):
    tb = x_ref.shape[0]
    logits = jax.lax.dot_general(
        w_ref[...], x_ref[...],
        dimension_numbers=(((1,), (1,)), ((), ())),
        preferred_element_type=jnp.float32,
        precision=jax.lax.Precision.DEFAULT,
    )
    scores = jax.nn.sigmoid(logits)
    sfc = scores + bias_ref[...]

    io_gs = lax.broadcasted_iota(jnp.int32, (GROUP_SIZE, tb), 0)
    io8 = lax.broadcasted_iota(jnp.int32, (N_GROUP, tb), 0)
    io64 = lax.broadcasted_iota(jnp.int32, (N_EXPERTS, tb), 0)

    rows = []
    for g in range(N_GROUP):
        blk = sfc[g * GROUP_SIZE:(g + 1) * GROUP_SIZE, :]
        m1, am = _first_argmax0(blk, io_gs, GROUP_SIZE)
        m2 = jnp.max(jnp.where(io_gs == am, NEG_INF, blk), axis=0, keepdims=True)
        rows.append(m1 + m2)
    group_scores = jnp.concatenate(rows, axis=0)

    gmask = jnp.zeros((N_GROUP, tb), dtype=jnp.bool_)
    gs = group_scores
    for _ in range(TOPK_GROUP):
        _, am = _first_argmax0(gs, io8, N_GROUP)
        sel = io8 == am
        gmask = jnp.logical_or(gmask, sel)
        gs = jnp.where(sel, NEG_INF, gs)

    blocks = [
        jnp.where(gmask[g:g + 1, :], sfc[g * GROUP_SIZE:(g + 1) * GROUP_SIZE, :], NEG_INF)
        for g in range(N_GROUP)
    ]
    tmp = jnp.concatenate(blocks, axis=0)
    idx_rows = []
    w_rows = []
    for _ in range(TOP_K):
        _, am = _first_argmax0(tmp, io64, N_EXPERTS)
        sel = io64 == am
        idx_rows.append(am)
        w_rows.append(jnp.max(jnp.where(sel, scores, NEG_INF), axis=0, keepdims=True))
        tmp = jnp.where(sel, NEG_INF, tmp)
    topk_idx = jnp.concatenate(idx_rows, axis=0)
    topk_w = jnp.concatenate(w_rows, axis=0)

    denom = jnp.sum(topk_w, axis=0, keepdims=True) + 1e-20
    idx_ref[...] = topk_idx
    w_out_ref[...] = topk_w / denom * SCALE


@jax.jit
def _run(x, weight, bias):
    t = x.shape[0]
    tb = 512
    grid = (t // tb,)
    return pl.pallas_call(
        _router_kernel,
        grid=grid,
        in_specs=[
            pl.BlockSpec((tb, x.shape[1]), lambda i: (i, 0)),
            pl.BlockSpec((N_EXPERTS, x.shape[1]), lambda i: (0, 0)),
            pl.BlockSpec((N_EXPERTS, 1), lambda i: (0, 0)),
        ],
        out_specs=[
            pl.BlockSpec((TOP_K, tb), lambda i: (0, i)),
            pl.BlockSpec((TOP_K, tb), lambda i: (0, i)),
        ],
        out_shape=[
            jax.ShapeDtypeStruct((TOP_K, t), jnp.int32),
            jax.ShapeDtypeStruct((TOP_K, t), jnp.float32),
        ],
    )(x, weight, bias)


def kernel(hidden_states, weight, e_score_correction_bias):
    bsz, seq_len, h = hidden_states.shape
    x = hidden_states.reshape(-1, h).astype(jnp.float32)
    bias = e_score_correction_bias.reshape(N_EXPERTS, 1).astype(jnp.float32)
    idx_t, w_t = _run(x, weight.astype(jnp.float32), bias)
    return idx_t.T, w_t.T


# speedup vs baseline: 5.7866x; 1.1389x over previous
import jax
import jax.numpy as jnp
from jax import lax
from jax.experimental import pallas as pl

N_EXPERTS = 64
N_GROUP = 8
GROUP_SIZE = N_EXPERTS // N_GROUP
TOPK_GROUP = 4
TOP_K = 8
SCALE = 2.5
NEG_INF = float("-inf")


def _first_argmax0(x, row_iota, height):
    m = jnp.max(x, axis=0, keepdims=True)
    am = jnp.min(jnp.where(x == m, row_iota, height), axis=0, keepdims=True)
    return m, am


def _router_kernel(x_ref, w_ref, bias_ref, idx_ref, w_out_ref):
    tb = x_ref.shape[0]
    logits = jax.lax.dot_general(
        w_ref[...], x_ref[...],
        dimension_numbers=(((1,), (1,)), ((), ())),
        preferred_element_type=jnp.float32,
        precision=jax.lax.Precision.DEFAULT,
    )
    scores = jax.nn.sigmoid(logits)
    sfc = scores + bias_ref[...]

    io_gs = lax.broadcasted_iota(jnp.int32, (GROUP_SIZE, tb), 0)
    io8 = lax.broadcasted_iota(jnp.int32, (N_GROUP, tb), 0)
    io64 = lax.broadcasted_iota(jnp.int32, (N_EXPERTS, tb), 0)

    rows = []
    for g in range(N_GROUP):
        blk = sfc[g * GROUP_SIZE:(g + 1) * GROUP_SIZE, :]
        m1, am = _first_argmax0(blk, io_gs, GROUP_SIZE)
        m2 = jnp.max(jnp.where(io_gs == am, NEG_INF, blk), axis=0, keepdims=True)
        rows.append(m1 + m2)
    group_scores = jnp.concatenate(rows, axis=0)

    gmask = jnp.zeros((N_GROUP, tb), dtype=jnp.bool_)
    gs = group_scores
    for _ in range(TOPK_GROUP):
        _, am = _first_argmax0(gs, io8, N_GROUP)
        sel = io8 == am
        gmask = jnp.logical_or(gmask, sel)
        gs = jnp.where(sel, NEG_INF, gs)

    blocks = [
        jnp.where(gmask[g:g + 1, :], sfc[g * GROUP_SIZE:(g + 1) * GROUP_SIZE, :], NEG_INF)
        for g in range(N_GROUP)
    ]
    tmp = jnp.concatenate(blocks, axis=0)
    idx_rows = []
    w_rows = []
    for _ in range(TOP_K):
        _, am = _first_argmax0(tmp, io64, N_EXPERTS)
        sel = io64 == am
        idx_rows.append(am)
        w_rows.append(jnp.max(jnp.where(sel, scores, NEG_INF), axis=0, keepdims=True))
        tmp = jnp.where(sel, NEG_INF, tmp)
    topk_idx = jnp.concatenate(idx_rows, axis=0)
    topk_w = jnp.concatenate(w_rows, axis=0)

    denom = jnp.sum(topk_w, axis=0, keepdims=True) + 1e-20
    idx_ref[...] = topk_idx
    w_out_ref[...] = topk_w / denom * SCALE


@jax.jit
def _run(x, weight, bias):
    t = x.shape[0]
    tb = 1024
    grid = (t // tb,)
    return pl.pallas_call(
        _router_kernel,
        grid=grid,
        in_specs=[
            pl.BlockSpec((tb, x.shape[1]), lambda i: (i, 0)),
            pl.BlockSpec((N_EXPERTS, x.shape[1]), lambda i: (0, 0)),
            pl.BlockSpec((N_EXPERTS, 1), lambda i: (0, 0)),
        ],
        out_specs=[
            pl.BlockSpec((TOP_K, tb), lambda i: (0, i)),
            pl.BlockSpec((TOP_K, tb), lambda i: (0, i)),
        ],
        out_shape=[
            jax.ShapeDtypeStruct((TOP_K, t), jnp.int32),
            jax.ShapeDtypeStruct((TOP_K, t), jnp.float32),
        ],
    )(x, weight, bias)


def kernel(hidden_states, weight, e_score_correction_bias):
    bsz, seq_len, h = hidden_states.shape
    x = hidden_states.reshape(-1, h).astype(jnp.float32)
    bias = e_score_correction_bias.reshape(N_EXPERTS, 1).astype(jnp.float32)
    idx_t, w_t = _run(x, weight.astype(jnp.float32), bias)
    return idx_t.T, w_t.T
